# Initial kernel scaffold; baseline (speedup 1.0000x reference)
import jax
import jax.numpy as jnp
from jax.experimental import pallas as pl


def kernel(x, edge_index, W1, b1, W2, b2, W3, b3, W4, b4, W5, b5):
    raise NotImplementedError("write your pallas kernel here")


# speedup vs baseline: 2.9558x; 2.9558x over previous
import jax
import jax.numpy as jnp
from jax import lax
from jax.experimental import pallas as pl
from jax.experimental.pallas import tpu as pltpu
from jax.experimental.pallas import tpu_sc as plsc

N_NODES = 10000
N_EDGES = 320000
NP = 10240
NC = 2
NS = 16
ROWS_PER_TILE = NP // NS
CH = 128
N1 = 80
N2 = 160
ZR = 160


def _make_sc_agg(n_chunks):
    mesh = plsc.VectorSubcoreMesh(core_axis_name="c", subcore_axis_name="s")

    nh = n_chunks // 2

    def body(src_hbm, sidx_hbm, didx_hbm, out_hbm,
             sidx_v, didx_v, rows_v, acc_sh, sem):
        cid = lax.axis_index("c")
        sid = lax.axis_index("s")

        def _zrow(r, c):
            for k in range(8):
                rows_v[r, pl.ds(16 * k, 16)] = jnp.zeros((16,), jnp.float32)
            return c
        lax.fori_loop(0, CH, _zrow, 0)
        base = sid * ROWS_PER_TILE

        def _zcp(i, c):
            pltpu.sync_copy(rows_v, acc_sh.at[pl.ds(base + i * CH, CH)])
            return c
        lax.fori_loop(0, ROWS_PER_TILE // CH, _zcp, 0)
        plsc.subcore_barrier()

        for half in range(2):
            pltpu.sync_copy(sidx_hbm.at[cid, sid, pl.ds(half * nh, nh)],
                            sidx_v)
            pltpu.sync_copy(didx_hbm.at[cid, sid, pl.ds(half * nh, nh)],
                            didx_v)

            def _step(j, c):
                pltpu.async_copy(src_hbm.at[sidx_v.at[j]], rows_v, sem).wait()
                pltpu.sync_copy(rows_v, acc_sh.at[didx_v.at[j]], add=True)
                return c
            lax.fori_loop(0, nh, _step, 0)
        plsc.subcore_barrier()

        pltpu.sync_copy(acc_sh.at[pl.ds(base, ROWS_PER_TILE)],
                        out_hbm.at[cid, pl.ds(base, ROWS_PER_TILE)])

    return pl.kernel(
        body, mesh=mesh,
        out_type=jax.ShapeDtypeStruct((NC, NP, 128), jnp.float32),
        scratch_types=[
            pltpu.VMEM((nh, CH), jnp.int32),
            pltpu.VMEM((nh, CH), jnp.int32),
            pltpu.VMEM((CH, 128), jnp.float32),
            pltpu.VMEM_SHARED((NP, 128), jnp.float32),
            pltpu.SemaphoreType.DMA,
        ])


_sc_agg_cache = {}


def _sc_agg(n_chunks, src_arr, sidx, didx):
    if n_chunks not in _sc_agg_cache:
        _sc_agg_cache[n_chunks] = _make_sc_agg(n_chunks)
    return _sc_agg_cache[n_chunks](src_arr, sidx, didx)


BN = 512
GRID = NP // BN


def _wspec(shape):
    return pl.BlockSpec(shape, lambda i: (0,) * len(shape))


def _mlp_first(x_pad, acc, W1, b1, W2, b2):
    def body(x_ref, a_ref, w1_ref, b1_ref, w2_ref, b2_ref, o_ref):
        g = x_ref[...] + a_ref[0] + a_ref[1]
        t = jnp.dot(g, w1_ref[...], preferred_element_type=jnp.float32)
        t = jnp.maximum(t + b1_ref[...], 0.0)
        h = jnp.dot(t, w2_ref[...], preferred_element_type=jnp.float32)
        h = jnp.maximum(h + b2_ref[...], 0.0)
        o_ref[0] = h[:, :128]
        o_ref[1] = h[:, 128:]

    return pl.pallas_call(
        body,
        grid=(GRID,),
        in_specs=[
            pl.BlockSpec((BN, 128), lambda i: (i, 0)),
            pl.BlockSpec((2, BN, 128), lambda i: (0, i, 0)),
            _wspec((128, 256)), _wspec((1, 256)),
            _wspec((256, 256)), _wspec((1, 256)),
        ],
        out_specs=pl.BlockSpec((2, BN, 128), lambda i: (0, i, 0)),
        out_shape=jax.ShapeDtypeStruct((2, NP, 128), jnp.float32),
    )(x_pad, acc, W1, b1, W2, b2)


def _mlp_mid(h, acc, W3, b3, W4, b4):
    def body(h_ref, a_ref, w3_ref, b3_ref, w4_ref, b4_ref, o_ref):
        g = jnp.concatenate([h_ref[0] + a_ref[0], h_ref[1] + a_ref[1]], axis=1)
        t = jnp.dot(g, w3_ref[...], preferred_element_type=jnp.float32)
        t = jnp.maximum(t + b3_ref[...], 0.0)
        hh = jnp.dot(t, w4_ref[...], preferred_element_type=jnp.float32)
        hh = jnp.maximum(hh + b4_ref[...], 0.0)
        o_ref[0] = hh[:, :128]
        o_ref[1] = hh[:, 128:]

    return pl.pallas_call(
        body,
        grid=(GRID,),
        in_specs=[
            pl.BlockSpec((2, BN, 128), lambda i: (0, i, 0)),
            pl.BlockSpec((2, BN, 128), lambda i: (0, i, 0)),
            _wspec((256, 256)), _wspec((1, 256)),
            _wspec((256, 256)), _wspec((1, 256)),
        ],
        out_specs=pl.BlockSpec((2, BN, 128), lambda i: (0, i, 0)),
        out_shape=jax.ShapeDtypeStruct((2, NP, 128), jnp.float32),
    )(h, acc, W3, b3, W4, b4)


def _mlp_last(h, acc, W5, b5):
    def body(h_ref, a_ref, w5_ref, b5_ref, o_ref):
        g = jnp.concatenate([h_ref[0] + a_ref[0], h_ref[1] + a_ref[1]], axis=1)
        o_ref[...] = jnp.dot(
            g, w5_ref[...], preferred_element_type=jnp.float32) + b5_ref[...]

    return pl.pallas_call(
        body,
        grid=(GRID,),
        in_specs=[
            pl.BlockSpec((2, BN, 128), lambda i: (0, i, 0)),
            pl.BlockSpec((2, BN, 128), lambda i: (0, i, 0)),
            _wspec((256, 128)), _wspec((1, 128)),
        ],
        out_specs=pl.BlockSpec((BN, 128), lambda i: (i, 0)),
        out_shape=jax.ShapeDtypeStruct((NP, 128), jnp.float32),
    )(h, acc, W5, b5)


def kernel(x, edge_index, W1, b1, W2, b2, W3, b3, W4, b4, W5, b5):
    src = edge_index[0].astype(jnp.int32)
    dst = edge_index[1].astype(jnp.int32)

    e1 = NC * NS * N1 * CH
    sidx1 = jnp.concatenate(
        [src, jnp.zeros((e1 - N_EDGES,), jnp.int32)]).reshape(NC, NS, N1, CH)
    didx1 = jnp.concatenate(
        [dst, jnp.full((e1 - N_EDGES,), N_NODES, jnp.int32)]
    ).reshape(NC, NS, N1, CH)

    e2 = NS * N2 * CH
    s2 = jnp.concatenate(
        [src, jnp.zeros((e2 - N_EDGES,), jnp.int32)]).reshape(1, NS, N2, CH)
    sidx2 = s2 + (jnp.arange(NC, dtype=jnp.int32) * NP).reshape(NC, 1, 1, 1)
    d2 = jnp.concatenate(
        [dst, jnp.full((e2 - N_EDGES,), N_NODES, jnp.int32)]
    ).reshape(1, NS, N2, CH)
    didx2 = jnp.broadcast_to(d2, (NC, NS, N2, CH))

    x_pad = jnp.pad(x, ((0, NP - N_NODES), (0, 0)))
    b1r, b2r, b3r, b4r, b5r = (
        b.reshape(1, -1) for b in (b1, b2, b3, b4, b5))

    acc1 = _sc_agg(N1, x, sidx1, didx1)
    h1 = _mlp_first(x_pad, acc1, W1, b1r, W2, b2r)
    acc2 = _sc_agg(N2, h1.reshape(2 * NP, 128), sidx2, didx2)
    h2 = _mlp_mid(h1, acc2, W3, b3r, W4, b4r)
    acc3 = _sc_agg(N2, h2.reshape(2 * NP, 128), sidx2, didx2)
    out = _mlp_last(h2, acc3, W5, b5r)
    return out[:N_NODES]
